# baseline (speedup 1.0000x reference)
import jax
import jax.numpy as jnp
from jax import lax
from jax.experimental import pallas as pl
from jax.experimental.pallas import tpu as pltpu
from jax.experimental.pallas import tpu_sc as plsc

EMBED_DIM = 64
BATCH = 16384
NROWS = 100000

NC = 2
NS = 16
LANES = 16
NW = NC * NS
B_PER_W = BATCH // NW
CHUNK = 128
NCHUNK = B_PER_W // CHUNK
NBUF = 2
PAIR_W = 2 * EMBED_DIM
NPAIR = NROWS // 2

SLAB = 256
NFULL = NROWS // SLAB
TAIL_R0 = NFULL * SLAB
TAIL_PAIRS = (NROWS - TAIL_R0) // 2
SLABS_PER_W = -(-NFULL // NW)


def _transpose_body(ut_hbm, it_hbm, ut_tail_hbm, it_tail_hbm,
                    uot_hbm, iot_hbm, slab0_v, slab1_v, blk0_v, blk1_v,
                    tail_v, isem0, isem1, osem0, osem1, sem):
    wid = lax.axis_index("s") * NC + lax.axis_index("c")
    lane = lax.broadcasted_iota(jnp.int32, (LANES,), 0)
    rots = [(lane + t) % LANES for t in range(LANES)]
    lconsts = [lane * SLAB + r for r in rots]
    sconsts = [lane + (r >> 1) * PAIR_W + ((r & 1) << 6) for r in rots]

    def transpose_blocks(slab, blk, w):
        def blk_step(b, c2):
            c0 = (b // (w // LANES)) * LANES
            j0 = (b % (w // LANES)) * LANES
            cvec = c0 + lane
            pr0 = j0 >> 1
            for t0 in range(0, LANES, 8):
                vs = [plsc.load_gather(slab, [cvec, j0 + rots[t]])
                      for t in range(t0, t0 + 8)]
                for i, t in enumerate(range(t0, t0 + 8)):
                    rot = rots[t]
                    plsc.store_scatter(blk, [pr0 + (rot >> 1),
                                             cvec + ((rot & 1) << 6)], vs[i])
            return c2

        lax.fori_loop(0, (EMBED_DIM // LANES) * (w // LANES), blk_step, 0)

    for tab_hbm, out_hbm in ((ut_hbm, uot_hbm), (it_hbm, iot_hbm)):
        slabs = ((slab0_v, blk0_v, isem0, osem0),
                 (slab1_v, blk1_v, isem1, osem1))

        def src(k):
            r0 = pl.multiple_of(k * SLAB, SLAB)
            return tab_hbm.at[:, pl.ds(r0, SLAB)]

        def dst(k):
            pr = pl.multiple_of(k * (SLAB // 2), SLAB // 2)
            return out_hbm.at[pl.ds(pr, SLAB // 2)]

        def fire_in(k, j):
            @pl.when(k < NFULL)
            def _():
                pltpu.async_copy(src(k), slabs[j][0], slabs[j][2])

        fire_in(wid, 0)

        def pair_step(m, carry):
            for j in range(2):
                i = 2 * m + j
                k = wid + i * NW
                slab, blk, isem, osem = slabs[j]

                @pl.when(k < NFULL)
                def _(_k=k, _slab=slab, _blk=blk, _isem=isem, _osem=osem,
                      _j=j, _i=i):
                    fire_in(_k + NW, 1 - _j)
                    @pl.when(_i >= 2)
                    def _():
                        pltpu.make_async_copy(_blk, dst(_k - 2 * NW),
                                              _osem).wait()
                    pltpu.make_async_copy(src(_k), _slab, _isem).wait()
                    transpose_blocks(_slab, _blk, SLAB)
                    pltpu.async_copy(_blk, dst(_k), _osem)

            return carry

        lax.fori_loop(0, (SLABS_PER_W + 1) // 2, pair_step, 0)

        pltpu.make_async_copy(blk0_v, dst(wid), osem0).wait()
        pltpu.make_async_copy(blk1_v, dst(wid), osem1).wait()

    for tail_w, (tail_hbm, out_hbm) in enumerate(
            ((ut_tail_hbm, uot_hbm), (it_tail_hbm, iot_hbm))):

        @pl.when(wid == tail_w)
        def _(_tail=tail_hbm, _out=out_hbm):
            pltpu.sync_copy(_tail, tail_v)
            pltpu.sync_copy(tail_v, _out.at[pl.ds(TAIL_R0 // 2, TAIL_PAIRS)])


SCHUNK = 64
NSTAGE = B_PER_W // SCHUNK
SNBUF = 4


def _score_body(upair_hbm, ppair_hbm, npair_hbm, uhalf_hbm, phalf_hbm,
                nhalf_hbm, utab_hbm, itab_hbm,
                pos_hbm, neg_hbm,
                uiv, piv, niv, uhv, phv, nhv,
                ub0, ub1, ub2, ub3, pb0, pb1, pb2, pb3, nb0, nb1, nb2, nb3,
                pos_v, neg_v, isem, sem0, sem1, sem2, sem3):
    wid = lax.axis_index("s") * NC + lax.axis_index("c")
    base = wid * B_PER_W
    k0 = wid * NCHUNK
    sems = (sem0, sem1, sem2, sem3)

    icopies = []
    for s in range(NCHUNK):
        icopies.append(pltpu.async_copy(upair_hbm.at[k0 + s], uiv.at[s], isem))
        icopies.append(pltpu.async_copy(ppair_hbm.at[k0 + s], piv.at[s], isem))
        icopies.append(pltpu.async_copy(npair_hbm.at[k0 + s], niv.at[s], isem))
        icopies.append(pltpu.async_copy(uhalf_hbm.at[k0 + s], uhv.at[s], isem))
        icopies.append(pltpu.async_copy(phalf_hbm.at[k0 + s], phv.at[s], isem))
        icopies.append(pltpu.async_copy(nhalf_hbm.at[k0 + s], nhv.at[s], isem))
    for cp in icopies:
        cp.wait()

    bufs = ((ub0, pb0, nb0), (ub1, pb1, nb1),
            (ub2, pb2, nb2), (ub3, pb3, nb3))

    def idx_slice(ref, s):
        return ref.at[s >> 1, pl.ds((s & 1) * SCHUNK, SCHUNK)]

    def fire(s):
        ub, pb, nb = bufs[s % SNBUF]
        sem = sems[s % SNBUF]
        return (pltpu.async_copy(utab_hbm.at[idx_slice(uiv, s)], ub, sem),
                pltpu.async_copy(itab_hbm.at[idx_slice(piv, s)], pb, sem),
                pltpu.async_copy(itab_hbm.at[idx_slice(niv, s)], nb, sem))

    lane = lax.broadcasted_iota(jnp.int32, (LANES,), 0)
    zero = jnp.zeros((LANES,), jnp.float32)
    NACC = 4

    pipeline = [fire(s) for s in range(SNBUF - 1)]
    for s in range(NSTAGE):
        for cp in pipeline.pop(0):
            cp.wait()
        if s + SNBUF - 1 < NSTAGE:
            pipeline.append(fire(s + SNBUF - 1))
        ub, pb, nb = bufs[s % SNBUF]

        def gbody(g, carry, _ub=ub, _pb=pb, _nb=nb, _s=s):
            row = g * LANES + lane
            hoff = (_s & 1) * SCHUNK + g * LANES
            hu = uhv[_s >> 1, pl.ds(hoff, LANES)]
            hp = phv[_s >> 1, pl.ds(hoff, LANES)]
            hn = nhv[_s >> 1, pl.ds(hoff, LANES)]
            paccs = [zero] * NACC
            naccs = [zero] * NACC
            for d in range(EMBED_DIM):
                col = (lane + d) & (EMBED_DIM - 1)
                lu = plsc.load_gather(_ub, [row, col + hu])
                lp = plsc.load_gather(_pb, [row, col + hp])
                ln = plsc.load_gather(_nb, [row, col + hn])
                k = d % NACC
                paccs[k] = paccs[k] + lu * lp
                naccs[k] = naccs[k] + lu * ln
            pacc = (paccs[0] + paccs[1]) + (paccs[2] + paccs[3])
            nacc = (naccs[0] + naccs[1]) + (naccs[2] + naccs[3])
            off = _s * SCHUNK + g * LANES
            pos_v[pl.ds(off, LANES)] = pacc
            neg_v[pl.ds(off, LANES)] = nacc
            return carry

        lax.fori_loop(0, SCHUNK // LANES, gbody, 0)

    pltpu.sync_copy(pos_v, pos_hbm.at[pl.ds(base, B_PER_W)])
    pltpu.sync_copy(neg_v, neg_hbm.at[pl.ds(base, B_PER_W)])


@jax.jit
def kernel(user_inputs, pos_item_inputs, neg_item_inputs, user_table, item_table):
    mesh = plsc.VectorSubcoreMesh(core_axis_name="c", subcore_axis_name="s")

    relayout = pl.kernel(
        _transpose_body,
        out_type=(jax.ShapeDtypeStruct((NPAIR, PAIR_W), jnp.float32),
                  jax.ShapeDtypeStruct((NPAIR, PAIR_W), jnp.float32)),
        mesh=mesh,
        compiler_params=pltpu.CompilerParams(needs_layout_passes=False, disable_bounds_checks=True),
        scratch_types=[
            pltpu.VMEM((EMBED_DIM, SLAB), jnp.float32),
            pltpu.VMEM((EMBED_DIM, SLAB), jnp.float32),
            pltpu.VMEM((SLAB // 2, PAIR_W), jnp.float32),
            pltpu.VMEM((SLAB // 2, PAIR_W), jnp.float32),
            pltpu.VMEM((TAIL_PAIRS, PAIR_W), jnp.float32),
            pltpu.SemaphoreType.DMA,
            pltpu.SemaphoreType.DMA,
            pltpu.SemaphoreType.DMA,
            pltpu.SemaphoreType.DMA,
            pltpu.SemaphoreType.DMA,
        ],
    )
    utab, itab = relayout(user_table.T, item_table.T,
                          user_table[TAIL_R0:].reshape(TAIL_PAIRS, PAIR_W),
                          item_table[TAIL_R0:].reshape(TAIL_PAIRS, PAIR_W))

    nrow = BATCH // CHUNK

    def prep(idx):
        idx = idx.astype(jnp.int32)
        return ((idx >> 1).reshape(nrow, CHUNK),
                ((idx & 1) << 6).reshape(nrow, CHUNK))

    upair, uhalf = prep(user_inputs)
    ppair, phalf = prep(pos_item_inputs)
    npair, nhalf = prep(neg_item_inputs)
    score = pl.kernel(
        _score_body,
        out_type=(jax.ShapeDtypeStruct((BATCH,), jnp.float32),
                  jax.ShapeDtypeStruct((BATCH,), jnp.float32)),
        mesh=mesh,
        compiler_params=pltpu.CompilerParams(needs_layout_passes=False, disable_bounds_checks=True),
        scratch_types=[
            pltpu.VMEM((NCHUNK, CHUNK), jnp.int32),
            pltpu.VMEM((NCHUNK, CHUNK), jnp.int32),
            pltpu.VMEM((NCHUNK, CHUNK), jnp.int32),
            pltpu.VMEM((NCHUNK, CHUNK), jnp.int32),
            pltpu.VMEM((NCHUNK, CHUNK), jnp.int32),
            pltpu.VMEM((NCHUNK, CHUNK), jnp.int32),
        ] + [pltpu.VMEM((SCHUNK, PAIR_W), jnp.float32)] * 12 + [
            pltpu.VMEM((B_PER_W,), jnp.float32),
            pltpu.VMEM((B_PER_W,), jnp.float32),
            pltpu.SemaphoreType.DMA,
            pltpu.SemaphoreType.DMA,
            pltpu.SemaphoreType.DMA,
            pltpu.SemaphoreType.DMA,
            pltpu.SemaphoreType.DMA,
        ],
    )
    return score(upair, ppair, npair, uhalf, phalf, nhalf, utab, itab)


# speedup vs baseline: 1.0343x; 1.0343x over previous
import jax
import jax.numpy as jnp
from jax import lax
from jax.experimental import pallas as pl
from jax.experimental.pallas import tpu as pltpu
from jax.experimental.pallas import tpu_sc as plsc

EMBED_DIM = 64
BATCH = 16384
NROWS = 100000

NC = 2
NS = 16
LANES = 16
NW = NC * NS
B_PER_W = BATCH // NW
CHUNK = 128
NCHUNK = B_PER_W // CHUNK
NBUF = 2
PAIR_W = 2 * EMBED_DIM
NPAIR = NROWS // 2

SLAB = 256
NFULL = NROWS // SLAB
TAIL_R0 = NFULL * SLAB
TAIL_PAIRS = (NROWS - TAIL_R0) // 2
SLABS_PER_W = -(-NFULL // NW)


def _transpose_body(ut_hbm, it_hbm, ut_tail_hbm, it_tail_hbm,
                    uot_hbm, iot_hbm, slab0_v, slab1_v, blk0_v, blk1_v,
                    tail_v, isem0, isem1, osem0, osem1, sem):
    wid = lax.axis_index("s") * NC + lax.axis_index("c")
    lane = lax.broadcasted_iota(jnp.int32, (LANES,), 0)
    rots = [(lane + t) % LANES for t in range(LANES)]
    lconsts = [lane * SLAB + r for r in rots]
    sconsts = [lane + (r >> 1) * PAIR_W + ((r & 1) << 6) for r in rots]

    def transpose_blocks(slab, blk, w):
        def blk_step(b, c2):
            c0 = (b // (w // LANES)) * LANES
            j0 = (b % (w // LANES)) * LANES
            cvec = c0 + lane
            pr0 = j0 >> 1
            for t0 in range(0, LANES, 8):
                vs = [plsc.load_gather(slab, [cvec, j0 + rots[t]])
                      for t in range(t0, t0 + 8)]
                for i, t in enumerate(range(t0, t0 + 8)):
                    rot = rots[t]
                    plsc.store_scatter(blk, [pr0 + (rot >> 1),
                                             cvec + ((rot & 1) << 6)], vs[i])
            return c2

        lax.fori_loop(0, (EMBED_DIM // LANES) * (w // LANES), blk_step, 0)

    for tab_hbm, out_hbm in ((ut_hbm, uot_hbm), (it_hbm, iot_hbm)):
        slabs = ((slab0_v, blk0_v, isem0, osem0),
                 (slab1_v, blk1_v, isem1, osem1))

        def src(k):
            r0 = pl.multiple_of(k * SLAB, SLAB)
            return tab_hbm.at[:, pl.ds(r0, SLAB)]

        def dst(k):
            pr = pl.multiple_of(k * (SLAB // 2), SLAB // 2)
            return out_hbm.at[pl.ds(pr, SLAB // 2)]

        def fire_in(k, j):
            @pl.when(k < NFULL)
            def _():
                pltpu.async_copy(src(k), slabs[j][0], slabs[j][2])

        fire_in(wid, 0)

        def pair_step(m, carry):
            for j in range(2):
                i = 2 * m + j
                k = wid + i * NW
                slab, blk, isem, osem = slabs[j]

                @pl.when(k < NFULL)
                def _(_k=k, _slab=slab, _blk=blk, _isem=isem, _osem=osem,
                      _j=j, _i=i):
                    fire_in(_k + NW, 1 - _j)
                    @pl.when(_i >= 2)
                    def _():
                        pltpu.make_async_copy(_blk, dst(_k - 2 * NW),
                                              _osem).wait()
                    pltpu.make_async_copy(src(_k), _slab, _isem).wait()
                    transpose_blocks(_slab, _blk, SLAB)
                    pltpu.async_copy(_blk, dst(_k), _osem)

            return carry

        lax.fori_loop(0, (SLABS_PER_W + 1) // 2, pair_step, 0)

        pltpu.make_async_copy(blk0_v, dst(wid), osem0).wait()
        pltpu.make_async_copy(blk1_v, dst(wid), osem1).wait()

    for tail_w, (tail_hbm, out_hbm) in enumerate(
            ((ut_tail_hbm, uot_hbm), (it_tail_hbm, iot_hbm))):

        @pl.when(wid == tail_w)
        def _(_tail=tail_hbm, _out=out_hbm):
            pltpu.sync_copy(_tail, tail_v)
            pltpu.sync_copy(tail_v, _out.at[pl.ds(TAIL_R0 // 2, TAIL_PAIRS)])


def _score_body(upair_hbm, ppair_hbm, npair_hbm, uhalf_hbm, phalf_hbm,
                nhalf_hbm, utab_hbm, itab_hbm,
                pos_hbm, neg_hbm,
                uiv, piv, niv, uhv, phv, nhv, ub0, ub1, pb0, pb1, nb0, nb1,
                pos_v, neg_v, isem, sem0, sem1):
    wid = lax.axis_index("s") * NC + lax.axis_index("c")
    base = wid * B_PER_W
    k0 = wid * NCHUNK
    sems = (sem0, sem1)

    icopies = []
    for s in range(NCHUNK):
        icopies.append(pltpu.async_copy(upair_hbm.at[k0 + s], uiv.at[s], isem))
        icopies.append(pltpu.async_copy(ppair_hbm.at[k0 + s], piv.at[s], isem))
        icopies.append(pltpu.async_copy(npair_hbm.at[k0 + s], niv.at[s], isem))
        icopies.append(pltpu.async_copy(uhalf_hbm.at[k0 + s], uhv.at[s], isem))
        icopies.append(pltpu.async_copy(phalf_hbm.at[k0 + s], phv.at[s], isem))
        icopies.append(pltpu.async_copy(nhalf_hbm.at[k0 + s], nhv.at[s], isem))
    for cp in icopies:
        cp.wait()

    bufs = ((ub0, pb0, nb0), (ub1, pb1, nb1))

    def fire(s):
        ub, pb, nb = bufs[s % NBUF]
        sem = sems[s % NBUF]
        return (pltpu.async_copy(utab_hbm.at[uiv.at[s]], ub, sem),
                pltpu.async_copy(itab_hbm.at[piv.at[s]], pb, sem),
                pltpu.async_copy(itab_hbm.at[niv.at[s]], nb, sem))

    lane = lax.broadcasted_iota(jnp.int32, (LANES,), 0)
    zero = jnp.zeros((LANES,), jnp.float32)
    NACC = 4

    inflight = fire(0)
    for s in range(NCHUNK):
        for cp in inflight:
            cp.wait()
        if s + 1 < NCHUNK:
            inflight = fire(s + 1)
        ub, pb, nb = bufs[s % NBUF]

        def gbody(g, carry, _ub=ub, _pb=pb, _nb=nb, _s=s):
            row = g * LANES + lane
            hu = uhv[_s, pl.ds(g * LANES, LANES)]
            hp = phv[_s, pl.ds(g * LANES, LANES)]
            hn = nhv[_s, pl.ds(g * LANES, LANES)]
            paccs = [zero] * NACC
            naccs = [zero] * NACC
            for d in range(EMBED_DIM):
                col = (lane + d) & (EMBED_DIM - 1)
                lu = plsc.load_gather(_ub, [row, col + hu])
                lp = plsc.load_gather(_pb, [row, col + hp])
                ln = plsc.load_gather(_nb, [row, col + hn])
                k = d % NACC
                paccs[k] = paccs[k] + lu * lp
                naccs[k] = naccs[k] + lu * ln
            pacc = (paccs[0] + paccs[1]) + (paccs[2] + paccs[3])
            nacc = (naccs[0] + naccs[1]) + (naccs[2] + naccs[3])
            off = _s * CHUNK + g * LANES
            pos_v[pl.ds(off, LANES)] = pacc
            neg_v[pl.ds(off, LANES)] = nacc
            return carry

        lax.fori_loop(0, CHUNK // LANES, gbody, 0)

    pltpu.sync_copy(pos_v, pos_hbm.at[pl.ds(base, B_PER_W)])
    pltpu.sync_copy(neg_v, neg_hbm.at[pl.ds(base, B_PER_W)])


@jax.jit
def kernel(user_inputs, pos_item_inputs, neg_item_inputs, user_table, item_table):
    mesh = plsc.VectorSubcoreMesh(core_axis_name="c", subcore_axis_name="s")

    relayout = pl.kernel(
        _transpose_body,
        out_type=(jax.ShapeDtypeStruct((NPAIR, PAIR_W), jnp.float32),
                  jax.ShapeDtypeStruct((NPAIR, PAIR_W), jnp.float32)),
        mesh=mesh,
        compiler_params=pltpu.CompilerParams(needs_layout_passes=False, disable_bounds_checks=True),
        scratch_types=[
            pltpu.VMEM((EMBED_DIM, SLAB), jnp.float32),
            pltpu.VMEM((EMBED_DIM, SLAB), jnp.float32),
            pltpu.VMEM((SLAB // 2, PAIR_W), jnp.float32),
            pltpu.VMEM((SLAB // 2, PAIR_W), jnp.float32),
            pltpu.VMEM((TAIL_PAIRS, PAIR_W), jnp.float32),
            pltpu.SemaphoreType.DMA,
            pltpu.SemaphoreType.DMA,
            pltpu.SemaphoreType.DMA,
            pltpu.SemaphoreType.DMA,
            pltpu.SemaphoreType.DMA,
        ],
    )
    utab, itab = relayout(user_table.T, item_table.T,
                          user_table[TAIL_R0:].reshape(TAIL_PAIRS, PAIR_W),
                          item_table[TAIL_R0:].reshape(TAIL_PAIRS, PAIR_W))

    nrow = BATCH // CHUNK

    def prep(idx):
        idx = idx.astype(jnp.int32)
        return ((idx >> 1).reshape(nrow, CHUNK),
                ((idx & 1) << 6).reshape(nrow, CHUNK))

    upair, uhalf = prep(user_inputs)
    ppair, phalf = prep(pos_item_inputs)
    npair, nhalf = prep(neg_item_inputs)
    score = pl.kernel(
        _score_body,
        out_type=(jax.ShapeDtypeStruct((BATCH,), jnp.float32),
                  jax.ShapeDtypeStruct((BATCH,), jnp.float32)),
        mesh=mesh,
        compiler_params=pltpu.CompilerParams(needs_layout_passes=False, disable_bounds_checks=True),
        scratch_types=[
            pltpu.VMEM((NCHUNK, CHUNK), jnp.int32),
            pltpu.VMEM((NCHUNK, CHUNK), jnp.int32),
            pltpu.VMEM((NCHUNK, CHUNK), jnp.int32),
            pltpu.VMEM((NCHUNK, CHUNK), jnp.int32),
            pltpu.VMEM((NCHUNK, CHUNK), jnp.int32),
            pltpu.VMEM((NCHUNK, CHUNK), jnp.int32),
            pltpu.VMEM((CHUNK, PAIR_W), jnp.float32),
            pltpu.VMEM((CHUNK, PAIR_W), jnp.float32),
            pltpu.VMEM((CHUNK, PAIR_W), jnp.float32),
            pltpu.VMEM((CHUNK, PAIR_W), jnp.float32),
            pltpu.VMEM((CHUNK, PAIR_W), jnp.float32),
            pltpu.VMEM((CHUNK, PAIR_W), jnp.float32),
            pltpu.VMEM((B_PER_W,), jnp.float32),
            pltpu.VMEM((B_PER_W,), jnp.float32),
            pltpu.SemaphoreType.DMA,
            pltpu.SemaphoreType.DMA,
            pltpu.SemaphoreType.DMA,
        ],
    )
    return score(upair, ppair, npair, uhalf, phalf, nhalf, utab, itab)


# speedup vs baseline: 1.0919x; 1.0557x over previous
import jax
import jax.numpy as jnp
from jax import lax
from jax.experimental import pallas as pl
from jax.experimental.pallas import tpu as pltpu
from jax.experimental.pallas import tpu_sc as plsc

EMBED_DIM = 64
BATCH = 16384
NROWS = 100000

NC = 2
NS = 16
LANES = 16
NW = NC * NS
B_PER_W = BATCH // NW
CHUNK = 128
NCHUNK = B_PER_W // CHUNK
NBUF = 2
PAIR_W = 2 * EMBED_DIM
NPAIR = NROWS // 2

SLAB = 256
NFULL = NROWS // SLAB
TAIL_R0 = NFULL * SLAB
TAIL_PAIRS = (NROWS - TAIL_R0) // 2
SLABS_PER_W = -(-NFULL // NW)


def _transpose_body(ut_hbm, it_hbm, ut_tail_hbm, it_tail_hbm,
                    uot_hbm, iot_hbm, slab0_v, slab1_v, blk0_v, blk1_v,
                    tail_v, isem0, isem1, osem0, osem1, sem):
    wid = lax.axis_index("s") * NC + lax.axis_index("c")
    lane = lax.broadcasted_iota(jnp.int32, (LANES,), 0)
    rots = [(lane + t) % LANES for t in range(LANES)]
    lconsts = [lane * SLAB + r for r in rots]
    sconsts = [lane + (r >> 1) * PAIR_W + ((r & 1) << 6) for r in rots]

    NCB = EMBED_DIM // LANES

    def transpose_blocks(slab, blk, b_lo, b_hi):
        def blk_step(b, c2):
            c0 = (b % NCB) * LANES
            j0 = (b // NCB) * LANES
            cvec = c0 + lane
            pr0 = j0 >> 1
            for t0 in range(0, LANES, 8):
                vs = [plsc.load_gather(slab, [cvec, j0 + rots[t]])
                      for t in range(t0, t0 + 8)]
                for i, t in enumerate(range(t0, t0 + 8)):
                    rot = rots[t]
                    plsc.store_scatter(blk, [pr0 + (rot >> 1),
                                             cvec + ((rot & 1) << 6)], vs[i])
            return c2

        lax.fori_loop(b_lo, b_hi, blk_step, 0)

    for tab_hbm, out_hbm in ((ut_hbm, uot_hbm), (it_hbm, iot_hbm)):
        slabs = ((slab0_v, blk0_v, isem0, osem0),
                 (slab1_v, blk1_v, isem1, osem1))

        def src(k):
            r0 = pl.multiple_of(k * SLAB, SLAB)
            return tab_hbm.at[:, pl.ds(r0, SLAB)]

        def dst(k):
            pr = pl.multiple_of(k * (SLAB // 2), SLAB // 2)
            return out_hbm.at[pl.ds(pr, SLAB // 2)]

        def fire_in(k, j):
            @pl.when(k < NFULL)
            def _():
                pltpu.async_copy(src(k), slabs[j][0], slabs[j][2])

        fire_in(wid, 0)

        def pair_step(m, carry):
            for j in range(2):
                i = 2 * m + j
                k = wid + i * NW
                slab, blk, isem, osem = slabs[j]

                @pl.when(k < NFULL)
                def _(_k=k, _slab=slab, _blk=blk, _isem=isem, _osem=osem,
                      _j=j, _i=i):
                    fire_in(_k + NW, 1 - _j)
                    @pl.when(_i >= 2)
                    def _():
                        pltpu.make_async_copy(_blk, dst(_k - 2 * NW),
                                              _osem).wait()
                    pltpu.make_async_copy(src(_k), _slab, _isem).wait()
                    nblk = NCB * (SLAB // LANES)
                    transpose_blocks(_slab, _blk, 0, nblk // 2)
                    pltpu.async_copy(_blk.at[pl.ds(0, SLAB // 4)],
                                     dst(_k).at[pl.ds(0, SLAB // 4)], _osem)
                    transpose_blocks(_slab, _blk, nblk // 2, nblk)
                    pltpu.async_copy(
                        _blk.at[pl.ds(SLAB // 4, SLAB // 4)],
                        dst(_k).at[pl.ds(SLAB // 4, SLAB // 4)], _osem)

            return carry

        lax.fori_loop(0, (SLABS_PER_W + 1) // 2, pair_step, 0)

        pltpu.make_async_copy(blk0_v, dst(wid), osem0).wait()
        pltpu.make_async_copy(blk1_v, dst(wid), osem1).wait()

    for tail_w, (tail_hbm, out_hbm) in enumerate(
            ((ut_tail_hbm, uot_hbm), (it_tail_hbm, iot_hbm))):

        @pl.when(wid == tail_w)
        def _(_tail=tail_hbm, _out=out_hbm):
            pltpu.sync_copy(_tail, tail_v)
            pltpu.sync_copy(tail_v, _out.at[pl.ds(TAIL_R0 // 2, TAIL_PAIRS)])


def _score_body(upair_hbm, ppair_hbm, npair_hbm, uhalf_hbm, phalf_hbm,
                nhalf_hbm, utab_hbm, itab_hbm,
                pos_hbm, neg_hbm,
                uiv, piv, niv, uhv, phv, nhv, ub0, ub1, pb0, pb1, nb0, nb1,
                pos_v, neg_v, isem, sem0, sem1):
    wid = lax.axis_index("s") * NC + lax.axis_index("c")
    base = wid * B_PER_W
    k0 = wid * NCHUNK
    sems = (sem0, sem1)

    icopies = []
    for s in range(NCHUNK):
        icopies.append(pltpu.async_copy(upair_hbm.at[k0 + s], uiv.at[s], isem))
        icopies.append(pltpu.async_copy(ppair_hbm.at[k0 + s], piv.at[s], isem))
        icopies.append(pltpu.async_copy(npair_hbm.at[k0 + s], niv.at[s], isem))
        icopies.append(pltpu.async_copy(uhalf_hbm.at[k0 + s], uhv.at[s], isem))
        icopies.append(pltpu.async_copy(phalf_hbm.at[k0 + s], phv.at[s], isem))
        icopies.append(pltpu.async_copy(nhalf_hbm.at[k0 + s], nhv.at[s], isem))
    for cp in icopies:
        cp.wait()

    bufs = ((ub0, pb0, nb0), (ub1, pb1, nb1))

    def fire(s):
        ub, pb, nb = bufs[s % NBUF]
        sem = sems[s % NBUF]
        return (pltpu.async_copy(utab_hbm.at[uiv.at[s]], ub, sem),
                pltpu.async_copy(itab_hbm.at[piv.at[s]], pb, sem),
                pltpu.async_copy(itab_hbm.at[niv.at[s]], nb, sem))

    lane = lax.broadcasted_iota(jnp.int32, (LANES,), 0)
    zero = jnp.zeros((LANES,), jnp.float32)
    NACC = 4

    inflight = fire(0)
    for s in range(NCHUNK):
        for cp in inflight:
            cp.wait()
        if s + 1 < NCHUNK:
            inflight = fire(s + 1)
        ub, pb, nb = bufs[s % NBUF]

        def gbody(g, carry, _ub=ub, _pb=pb, _nb=nb, _s=s):
            row = g * LANES + lane
            hu = uhv[_s, pl.ds(g * LANES, LANES)]
            hp = phv[_s, pl.ds(g * LANES, LANES)]
            hn = nhv[_s, pl.ds(g * LANES, LANES)]
            paccs = [zero] * NACC
            naccs = [zero] * NACC
            for d in range(EMBED_DIM):
                col = (lane + d) & (EMBED_DIM - 1)
                lu = plsc.load_gather(_ub, [row, col + hu])
                lp = plsc.load_gather(_pb, [row, col + hp])
                ln = plsc.load_gather(_nb, [row, col + hn])
                k = d % NACC
                paccs[k] = paccs[k] + lu * lp
                naccs[k] = naccs[k] + lu * ln
            pacc = (paccs[0] + paccs[1]) + (paccs[2] + paccs[3])
            nacc = (naccs[0] + naccs[1]) + (naccs[2] + naccs[3])
            off = _s * CHUNK + g * LANES
            pos_v[pl.ds(off, LANES)] = pacc
            neg_v[pl.ds(off, LANES)] = nacc
            return carry

        lax.fori_loop(0, CHUNK // LANES, gbody, 0)

    pltpu.sync_copy(pos_v, pos_hbm.at[pl.ds(base, B_PER_W)])
    pltpu.sync_copy(neg_v, neg_hbm.at[pl.ds(base, B_PER_W)])


@jax.jit
def kernel(user_inputs, pos_item_inputs, neg_item_inputs, user_table, item_table):
    mesh = plsc.VectorSubcoreMesh(core_axis_name="c", subcore_axis_name="s")

    relayout = pl.kernel(
        _transpose_body,
        out_type=(jax.ShapeDtypeStruct((NPAIR, PAIR_W), jnp.float32),
                  jax.ShapeDtypeStruct((NPAIR, PAIR_W), jnp.float32)),
        mesh=mesh,
        compiler_params=pltpu.CompilerParams(needs_layout_passes=False, disable_bounds_checks=True),
        scratch_types=[
            pltpu.VMEM((EMBED_DIM, SLAB), jnp.float32),
            pltpu.VMEM((EMBED_DIM, SLAB), jnp.float32),
            pltpu.VMEM((SLAB // 2, PAIR_W), jnp.float32),
            pltpu.VMEM((SLAB // 2, PAIR_W), jnp.float32),
            pltpu.VMEM((TAIL_PAIRS, PAIR_W), jnp.float32),
            pltpu.SemaphoreType.DMA,
            pltpu.SemaphoreType.DMA,
            pltpu.SemaphoreType.DMA,
            pltpu.SemaphoreType.DMA,
            pltpu.SemaphoreType.DMA,
        ],
    )
    utab, itab = relayout(user_table.T, item_table.T,
                          user_table[TAIL_R0:].reshape(TAIL_PAIRS, PAIR_W),
                          item_table[TAIL_R0:].reshape(TAIL_PAIRS, PAIR_W))

    nrow = BATCH // CHUNK

    def prep(idx):
        idx = idx.astype(jnp.int32)
        return ((idx >> 1).reshape(nrow, CHUNK),
                ((idx & 1) << 6).reshape(nrow, CHUNK))

    upair, uhalf = prep(user_inputs)
    ppair, phalf = prep(pos_item_inputs)
    npair, nhalf = prep(neg_item_inputs)
    score = pl.kernel(
        _score_body,
        out_type=(jax.ShapeDtypeStruct((BATCH,), jnp.float32),
                  jax.ShapeDtypeStruct((BATCH,), jnp.float32)),
        mesh=mesh,
        compiler_params=pltpu.CompilerParams(needs_layout_passes=False, disable_bounds_checks=True),
        scratch_types=[
            pltpu.VMEM((NCHUNK, CHUNK), jnp.int32),
            pltpu.VMEM((NCHUNK, CHUNK), jnp.int32),
            pltpu.VMEM((NCHUNK, CHUNK), jnp.int32),
            pltpu.VMEM((NCHUNK, CHUNK), jnp.int32),
            pltpu.VMEM((NCHUNK, CHUNK), jnp.int32),
            pltpu.VMEM((NCHUNK, CHUNK), jnp.int32),
            pltpu.VMEM((CHUNK, PAIR_W), jnp.float32),
            pltpu.VMEM((CHUNK, PAIR_W), jnp.float32),
            pltpu.VMEM((CHUNK, PAIR_W), jnp.float32),
            pltpu.VMEM((CHUNK, PAIR_W), jnp.float32),
            pltpu.VMEM((CHUNK, PAIR_W), jnp.float32),
            pltpu.VMEM((CHUNK, PAIR_W), jnp.float32),
            pltpu.VMEM((B_PER_W,), jnp.float32),
            pltpu.VMEM((B_PER_W,), jnp.float32),
            pltpu.SemaphoreType.DMA,
            pltpu.SemaphoreType.DMA,
            pltpu.SemaphoreType.DMA,
        ],
    )
    return score(upair, ppair, npair, uhalf, phalf, nhalf, utab, itab)
